# Initial kernel scaffold; baseline (speedup 1.0000x reference)
import jax
import jax.numpy as jnp
from jax.experimental import pallas as pl
from jax.experimental.pallas import tpu as pltpu

_NBR = 5
_NCOMP = 32
_BIG = 1e30
_THRESH = 1e29
_BM = 256
_BK = 2688


def _dist_body(x_ref, y_ref, out_ref, acc_ref, sqx_ref, sqy_ref):
    bk = pl.program_id(2)

    @pl.when(bk == 0)
    def _init():
        acc_ref[...] = jnp.zeros_like(acc_ref)
        sqx_ref[...] = jnp.zeros_like(sqx_ref)
        sqy_ref[...] = jnp.zeros_like(sqy_ref)

    x = x_ref[...]
    y = y_ref[...]
    acc_ref[...] += jax.lax.dot_general(
        x, y, (((1,), (1,)), ((), ())), preferred_element_type=jnp.float32)
    sqx_ref[...] += jnp.sum(x * x, axis=1, keepdims=True)
    sqy_ref[...] += jnp.sum(y * y, axis=1, keepdims=True)

    @pl.when(bk == pl.num_programs(2) - 1)
    def _fin():
        d2 = sqx_ref[...] + jnp.transpose(sqy_ref[...]) - 2.0 * acc_ref[...]
        out_ref[...] = jnp.sqrt(jnp.maximum(d2, 0.0))


def _graph_body(dist_ref, g_ref, a_ref, w_ref):
    n = dist_ref.shape[0]
    dist = dist_ref[...]
    w_ref[...] = dist
    a_ref[...] = jnp.full((n, n), _BIG, jnp.float32)
    colidx = jax.lax.broadcasted_iota(jnp.int32, (n, n), 1)

    for t in range(_NBR + 1):
        wv = w_ref[...]
        rowmin = jnp.min(wv, axis=1, keepdims=True)
        sel = wv == rowmin
        idx = jnp.min(jnp.where(sel, colidx, n), axis=1, keepdims=True)
        onehot = colidx == idx
        if t > 0:
            a_ref[...] = jnp.where(onehot, dist, a_ref[...])
        w_ref[...] = jnp.where(onehot, _BIG, wv)

    a = jnp.minimum(a_ref[...], jnp.transpose(a_ref[...]))
    eye = jax.lax.broadcasted_iota(jnp.int32, (n, n), 0) == colidx
    a_ref[...] = jnp.where(eye, 0.0, a)

    def fw(k, carry):
        rowk = a_ref[pl.ds(k, 1), :]
        colk = jnp.transpose(rowk)
        a_ref[...] = jnp.minimum(a_ref[...], colk + rowk)
        return carry

    jax.lax.fori_loop(0, n, fw, 0)

    d = a_ref[...]
    finite = d < _THRESH
    dmax = jnp.max(jnp.where(finite, d, 0.0))
    d = jnp.where(finite, d, dmax)
    d2 = d * d
    inv_n = jnp.float32(1.0 / n)
    s1 = jnp.sum(d2, axis=1, keepdims=True) * inv_n
    s0 = jnp.sum(d2, axis=0, keepdims=True) * inv_n
    gm = jnp.sum(s1) * inv_n
    g_ref[...] = -0.5 * (d2 - s0 - s1 + gm)


def kernel(toLearn):
    flat = toLearn.reshape(toLearn.shape[0], -1)
    n, dim = flat.shape
    nk = dim // _BK
    dist = pl.pallas_call(
        _dist_body,
        grid=(n // _BM, n // _BM, nk),
        in_specs=[
            pl.BlockSpec((_BM, _BK), lambda i, j, k: (i, k)),
            pl.BlockSpec((_BM, _BK), lambda i, j, k: (j, k)),
        ],
        out_specs=pl.BlockSpec((_BM, _BM), lambda i, j, k: (i, j)),
        out_shape=jax.ShapeDtypeStruct((n, n), jnp.float32),
        scratch_shapes=[
            pltpu.VMEM((_BM, _BM), jnp.float32),
            pltpu.VMEM((_BM, 1), jnp.float32),
            pltpu.VMEM((_BM, 1), jnp.float32),
        ],
        compiler_params=pltpu.CompilerParams(
            dimension_semantics=("arbitrary", "arbitrary", "arbitrary")),
    )(flat, flat)

    g = pl.pallas_call(
        _graph_body,
        out_shape=jax.ShapeDtypeStruct((n, n), jnp.float32),
        scratch_shapes=[
            pltpu.VMEM((n, n), jnp.float32),
            pltpu.VMEM((n, n), jnp.float32),
        ],
    )(dist)

    w, v = jnp.linalg.eigh(g)
    w = w[::-1][:_NCOMP]
    v = v[:, ::-1][:, :_NCOMP]
    emb = v * jnp.sqrt(jnp.maximum(w, 0.0))[None, :]
    return emb.astype(jnp.float32)


# speedup vs baseline: 1.0323x; 1.0323x over previous
import jax
import jax.numpy as jnp
from jax.experimental import pallas as pl
from jax.experimental.pallas import tpu as pltpu

_NBR = 5
_NCOMP = 32


def _graph_body(dist_ref, d_ref, w_ref):
    n = dist_ref.shape[0]
    inf = jnp.float32(jnp.inf)
    dist = dist_ref[...]
    w_ref[...] = dist
    d_ref[...] = jnp.full((n, n), inf, jnp.float32)
    colidx = jax.lax.broadcasted_iota(jnp.int32, (n, n), 1)

    for t in range(_NBR + 1):
        wv = w_ref[...]
        rowmin = jnp.min(wv, axis=1, keepdims=True)
        sel = wv == rowmin
        idx = jnp.min(jnp.where(sel, colidx, n), axis=1, keepdims=True)
        onehot = colidx == idx
        if t > 0:
            d_ref[...] = jnp.where(onehot, dist, d_ref[...])
        w_ref[...] = jnp.where(onehot, inf, wv)

    a = jnp.minimum(d_ref[...], jnp.transpose(d_ref[...]))
    eye = jax.lax.broadcasted_iota(jnp.int32, (n, n), 0) == colidx
    d_ref[...] = jnp.where(eye, 0.0, a)

    def fw(k, carry):
        rowk = d_ref[pl.ds(k, 1), :]
        colk = jnp.transpose(rowk)
        d_ref[...] = jnp.minimum(d_ref[...], colk + rowk)
        return carry

    jax.lax.fori_loop(0, n, fw, 0)


def kernel(toLearn):
    flat = toLearn.reshape(toLearn.shape[0], -1)
    n = flat.shape[0]
    sq = jnp.sum(flat * flat, axis=1)
    d2 = sq[:, None] + sq[None, :] - 2.0 * (flat @ flat.T)
    d2 = jnp.maximum(d2, 0.0)
    dist = jnp.sqrt(d2)

    D = pl.pallas_call(
        _graph_body,
        out_shape=jax.ShapeDtypeStruct((n, n), jnp.float32),
        scratch_shapes=[pltpu.VMEM((n, n), jnp.float32)],
    )(dist)

    finite = jnp.isfinite(D)
    dmax = jnp.max(jnp.where(finite, D, 0.0))
    D = jnp.where(finite, D, dmax)
    D2 = D * D
    G = -0.5 * (D2 - D2.mean(axis=0, keepdims=True)
                - D2.mean(axis=1, keepdims=True) + D2.mean())
    w, v = jnp.linalg.eigh(G)
    w = w[::-1][:_NCOMP]
    v = v[:, ::-1][:, :_NCOMP]
    emb = v * jnp.sqrt(jnp.maximum(w, 0.0))[None, :]
    return emb.astype(jnp.float32)


# speedup vs baseline: 1.0388x; 1.0063x over previous
import jax
import jax.numpy as jnp
from jax.experimental import pallas as pl
from jax.experimental.pallas import tpu as pltpu

_NBR = 5
_NCOMP = 32
_FWB = 8


def _graph_body(dist_ref, d_ref, w_ref):
    n = dist_ref.shape[0]
    inf = jnp.float32(jnp.inf)
    dist = dist_ref[...]
    w_ref[...] = dist
    d_ref[...] = jnp.full((n, n), inf, jnp.float32)
    colidx = jax.lax.broadcasted_iota(jnp.int32, (n, n), 1)

    for t in range(_NBR + 1):
        wv = w_ref[...]
        rowmin = jnp.min(wv, axis=1, keepdims=True)
        sel = wv == rowmin
        idx = jnp.min(jnp.where(sel, colidx, n), axis=1, keepdims=True)
        onehot = colidx == idx
        if t > 0:
            d_ref[...] = jnp.where(onehot, dist, d_ref[...])
        w_ref[...] = jnp.where(onehot, inf, wv)

    a = jnp.minimum(d_ref[...], jnp.transpose(d_ref[...]))
    eye = jax.lax.broadcasted_iota(jnp.int32, (n, n), 0) == colidx
    d_ref[...] = jnp.where(eye, 0.0, a)

    def fw_block(kb, carry):
        k0 = kb * _FWB
        p = d_ref[pl.ds(k0, _FWB), :]
        pd = pltpu.roll(p, -k0, 1)[:, :_FWB]
        snaps = []
        for t in range(_FWB):
            rowt = p[t:t + 1, :]
            snaps.append(rowt)
            colt = pd[:, t:t + 1]
            p = jnp.minimum(p, colt + rowt)
            pd = jnp.minimum(pd, colt + pd[t:t + 1, :])
        s = jnp.concatenate(snaps, axis=0)
        st = jnp.transpose(s)
        upd = st[:, 0:1] + s[0:1, :]
        for t in range(1, _FWB):
            upd = jnp.minimum(upd, st[:, t:t + 1] + s[t:t + 1, :])
        d_ref[...] = jnp.minimum(d_ref[...], upd)
        return carry

    jax.lax.fori_loop(0, n // _FWB, fw_block, 0)


def kernel(toLearn):
    flat = toLearn.reshape(toLearn.shape[0], -1)
    n = flat.shape[0]
    sq = jnp.sum(flat * flat, axis=1)
    d2 = sq[:, None] + sq[None, :] - 2.0 * (flat @ flat.T)
    d2 = jnp.maximum(d2, 0.0)
    dist = jnp.sqrt(d2)

    D = pl.pallas_call(
        _graph_body,
        out_shape=jax.ShapeDtypeStruct((n, n), jnp.float32),
        scratch_shapes=[pltpu.VMEM((n, n), jnp.float32)],
    )(dist)

    finite = jnp.isfinite(D)
    dmax = jnp.max(jnp.where(finite, D, 0.0))
    D = jnp.where(finite, D, dmax)
    D2 = D * D
    G = -0.5 * (D2 - D2.mean(axis=0, keepdims=True)
                - D2.mean(axis=1, keepdims=True) + D2.mean())
    w, v = jnp.linalg.eigh(G)
    w = w[::-1][:_NCOMP]
    v = v[:, ::-1][:, :_NCOMP]
    emb = v * jnp.sqrt(jnp.maximum(w, 0.0))[None, :]
    return emb.astype(jnp.float32)


# speedup vs baseline: 1.0405x; 1.0017x over previous
import jax
import jax.numpy as jnp
from jax.experimental import pallas as pl
from jax.experimental.pallas import tpu as pltpu

_NBR = 5
_NCOMP = 32
_FWB = 32


def _graph_body(dist_ref, d_ref, w_ref):
    n = dist_ref.shape[0]
    inf = jnp.float32(jnp.inf)
    dist = dist_ref[...]
    w_ref[...] = dist
    d_ref[...] = jnp.full((n, n), inf, jnp.float32)
    colidx = jax.lax.broadcasted_iota(jnp.int32, (n, n), 1)

    for t in range(_NBR + 1):
        wv = w_ref[...]
        rowmin = jnp.min(wv, axis=1, keepdims=True)
        sel = wv == rowmin
        idx = jnp.min(jnp.where(sel, colidx, n), axis=1, keepdims=True)
        onehot = colidx == idx
        if t > 0:
            d_ref[...] = jnp.where(onehot, dist, d_ref[...])
        w_ref[...] = jnp.where(onehot, inf, wv)

    a = jnp.minimum(d_ref[...], jnp.transpose(d_ref[...]))
    eye = jax.lax.broadcasted_iota(jnp.int32, (n, n), 0) == colidx
    d_ref[...] = jnp.where(eye, 0.0, a)

    def fw_block(kb, carry):
        k0 = kb * _FWB
        p = d_ref[pl.ds(k0, _FWB), :]
        pd = pltpu.roll(p, -k0, 1)[:, :_FWB]
        snaps = []
        for t in range(_FWB):
            rowt = p[t:t + 1, :]
            snaps.append(rowt)
            colt = pd[:, t:t + 1]
            p = jnp.minimum(p, colt + rowt)
            pd = jnp.minimum(pd, colt + pd[t:t + 1, :])
        s = jnp.concatenate(snaps, axis=0)
        st = jnp.transpose(s)
        upd = st[:, 0:1] + s[0:1, :]
        for t in range(1, _FWB):
            upd = jnp.minimum(upd, st[:, t:t + 1] + s[t:t + 1, :])
        d_ref[...] = jnp.minimum(d_ref[...], upd)
        return carry

    jax.lax.fori_loop(0, n // _FWB, fw_block, 0)


def kernel(toLearn):
    flat = toLearn.reshape(toLearn.shape[0], -1)
    n = flat.shape[0]
    sq = jnp.sum(flat * flat, axis=1)
    d2 = sq[:, None] + sq[None, :] - 2.0 * (flat @ flat.T)
    d2 = jnp.maximum(d2, 0.0)
    dist = jnp.sqrt(d2)

    D = pl.pallas_call(
        _graph_body,
        out_shape=jax.ShapeDtypeStruct((n, n), jnp.float32),
        scratch_shapes=[pltpu.VMEM((n, n), jnp.float32)],
    )(dist)

    finite = jnp.isfinite(D)
    dmax = jnp.max(jnp.where(finite, D, 0.0))
    D = jnp.where(finite, D, dmax)
    D2 = D * D
    G = -0.5 * (D2 - D2.mean(axis=0, keepdims=True)
                - D2.mean(axis=1, keepdims=True) + D2.mean())
    w, v = jnp.linalg.eigh(G)
    w = w[::-1][:_NCOMP]
    v = v[:, ::-1][:, :_NCOMP]
    emb = v * jnp.sqrt(jnp.maximum(w, 0.0))[None, :]
    return emb.astype(jnp.float32)
